# baseline (speedup 1.0000x reference)
import functools
import math

import jax
import jax.numpy as jnp
from jax import lax
from jax.experimental import pallas as pl
from jax.experimental.pallas import tpu as pltpu
from jax.experimental.pallas import tpu_sc as plsc

D_MODEL = 64
SCALE = math.sqrt(D_MODEL)

NUM_CORES = 2
NUM_SUBCORES = 16
NUM_WORKERS = NUM_CORES * NUM_SUBCORES
LANES = 16

GATHER_ROWS = 128
GATHERS_PER_CHUNK = 5
CHUNK_ROWS = GATHER_ROWS * GATHERS_PER_CHUNK
ROW_UNROLL = 8


@functools.lru_cache(maxsize=None)
def _build(B: int):
    assert B % (NUM_WORKERS * CHUNK_ROWS) == 0
    b_per_w = B // NUM_WORKERS
    k_per_w = b_per_w // GATHER_ROWS
    num_chunks = b_per_w // CHUNK_ROWS

    mesh = plsc.VectorSubcoreMesh(core_axis_name="c", subcore_axis_name="s")

    @functools.partial(
        pl.kernel,
        mesh=mesh,
        out_type=jax.ShapeDtypeStruct((B, D_MODEL), jnp.float32),
        scratch_types=[
            pltpu.VMEM((k_per_w, GATHER_ROWS), jnp.int32),
            pltpu.VMEM((CHUNK_ROWS, D_MODEL), jnp.float32),
            pltpu.VMEM((CHUNK_ROWS, D_MODEL), jnp.float32),
            pltpu.SemaphoreType.DMA,
            pltpu.SemaphoreType.DMA,
        ],
        compiler_params=pltpu.CompilerParams(use_tc_tiling_on_sc=False),
    )
    def emb_kernel(x_hbm, lut_hbm, out_hbm, idx_v, buf0, buf1, sem0, sem1):
        wid = lax.axis_index("s") * NUM_CORES + lax.axis_index("c")
        base = wid * b_per_w

        pltpu.sync_copy(x_hbm.at[wid], idx_v)

        bufs = (buf0, buf1)
        sems = (sem0, sem1)

        def fire(g, buf, sem):
            waits = []
            for j in range(GATHERS_PER_CHUNK):
                waits.append(
                    pltpu.async_copy(
                        lut_hbm.at[idx_v.at[g * GATHERS_PER_CHUNK + j]],
                        buf.at[pl.ds(j * GATHER_ROWS, GATHER_ROWS)],
                        sem,
                    )
                )
            return waits

        def scale(buf):
            def body(i, carry):
                for r in range(ROW_UNROLL):
                    row = i * ROW_UNROLL + r
                    for c in range(D_MODEL // LANES):
                        sl = pl.ds(c * LANES, LANES)
                        buf[row, sl] = buf[row, sl] * SCALE
                return carry

            lax.fori_loop(0, CHUNK_ROWS // ROW_UNROLL, body, 0)

        inflight = fire(0, bufs[0], sems[0])
        for g in range(num_chunks):
            cur = bufs[g % 2]
            nxt_inflight = (
                fire(g + 1, bufs[(g + 1) % 2], sems[(g + 1) % 2])
                if g + 1 < num_chunks
                else []
            )
            for w in inflight:
                w.wait()
            inflight = nxt_inflight
            scale(cur)
            pltpu.sync_copy(
                cur, out_hbm.at[pl.ds(base + g * CHUNK_ROWS, CHUNK_ROWS)]
            )

    return emb_kernel


def kernel(x, lut):
    batch, seq = x.shape
    B = batch * seq
    x_flat = jnp.reshape(x.astype(jnp.int32),
                         (NUM_WORKERS, B // (NUM_WORKERS * GATHER_ROWS),
                          GATHER_ROWS))
    out = _build(B)(x_flat, lut)
    return jnp.reshape(out, (batch, seq, D_MODEL))


# speedup vs baseline: 1.0023x; 1.0023x over previous
import functools
import math

import jax
import jax.numpy as jnp
from jax import lax
from jax.experimental import pallas as pl
from jax.experimental.pallas import tpu as pltpu
from jax.experimental.pallas import tpu_sc as plsc

D_MODEL = 64
SCALE = math.sqrt(D_MODEL)

NUM_CORES = 2
NUM_SUBCORES = 16
NUM_WORKERS = NUM_CORES * NUM_SUBCORES
LANES = 16

ROWS_PER_CHUNK = 4
IDX_SPLITS = ((0, 128), (128, 72))


@functools.lru_cache(maxsize=None)
def _build(batch: int, seq: int):
    rows_per_w = batch // NUM_WORKERS
    num_chunks = rows_per_w // ROWS_PER_CHUNK

    mesh = plsc.VectorSubcoreMesh(core_axis_name="c", subcore_axis_name="s")

    @functools.partial(
        pl.kernel,
        mesh=mesh,
        out_type=jax.ShapeDtypeStruct((batch, seq, D_MODEL), jnp.float32),
        scratch_types=[
            pltpu.VMEM((rows_per_w, seq), jnp.int32),
            pltpu.VMEM((ROWS_PER_CHUNK, seq, D_MODEL), jnp.float32),
            pltpu.VMEM((ROWS_PER_CHUNK, seq, D_MODEL), jnp.float32),
            pltpu.SemaphoreType.DMA,
            pltpu.SemaphoreType.DMA,
        ],
        compiler_params=pltpu.CompilerParams(use_tc_tiling_on_sc=False),
    )
    def emb_kernel(x_hbm, lut_hbm, out_hbm, idx_v, buf0, buf1, sem0, sem1):
        wid = lax.axis_index("s") * NUM_CORES + lax.axis_index("c")
        row0 = wid * rows_per_w

        pltpu.sync_copy(x_hbm.at[pl.ds(row0, rows_per_w)], idx_v)

        bufs = (buf0, buf1)
        sems = (sem0, sem1)

        def fire(g, buf, sem):
            waits = []
            for r in range(ROWS_PER_CHUNK):
                xrow = g * ROWS_PER_CHUNK + r
                for off, n in IDX_SPLITS:
                    waits.append(
                        pltpu.async_copy(
                            lut_hbm.at[idx_v.at[xrow, pl.ds(off, n)]],
                            buf.at[r, pl.ds(off, n)],
                            sem,
                        )
                    )
            return waits

        def scale(buf):
            def body(i, carry):
                s0 = i * 2
                for r in range(ROWS_PER_CHUNK):
                    for ds in range(2):
                        for c in range(D_MODEL // LANES):
                            sl = pl.ds(c * LANES, LANES)
                            buf[r, s0 + ds, sl] = buf[r, s0 + ds, sl] * SCALE
                return carry

            lax.fori_loop(0, seq // 2, body, 0)

        inflight = fire(0, bufs[0], sems[0])
        for g in range(num_chunks):
            cur = bufs[g % 2]
            nxt_inflight = (
                fire(g + 1, bufs[(g + 1) % 2], sems[(g + 1) % 2])
                if g + 1 < num_chunks
                else []
            )
            for w in inflight:
                w.wait()
            inflight = nxt_inflight
            scale(cur)
            pltpu.sync_copy(
                cur, out_hbm.at[pl.ds(row0 + g * ROWS_PER_CHUNK, ROWS_PER_CHUNK)]
            )

    return emb_kernel


def kernel(x, lut):
    batch, seq = x.shape
    return _build(batch, seq)(x.astype(jnp.int32), lut)


# speedup vs baseline: 1.4527x; 1.4494x over previous
import functools
import math

import jax
import jax.numpy as jnp
from jax import lax
from jax.experimental import pallas as pl
from jax.experimental.pallas import tpu as pltpu
from jax.experimental.pallas import tpu_sc as plsc

D_MODEL = 64
SCALE = math.sqrt(D_MODEL)

NUM_CORES = 2
NUM_SUBCORES = 16
NUM_WORKERS = NUM_CORES * NUM_SUBCORES
LANES = 16

ROWS_PER_CHUNK = 2


@functools.lru_cache(maxsize=None)
def _build(batch: int, seq: int):
    rows_per_w = batch // NUM_WORKERS
    num_chunks = rows_per_w // ROWS_PER_CHUNK

    mesh = plsc.VectorSubcoreMesh(core_axis_name="c", subcore_axis_name="s")

    @functools.partial(
        pl.kernel,
        mesh=mesh,
        out_type=jax.ShapeDtypeStruct((batch, seq, D_MODEL), jnp.float32),
        scratch_types=[
            pltpu.VMEM((rows_per_w, seq), jnp.int32),
            pltpu.VMEM((ROWS_PER_CHUNK, seq, D_MODEL), jnp.float32),
            pltpu.VMEM((ROWS_PER_CHUNK, seq, D_MODEL), jnp.float32),
            pltpu.SemaphoreType.DMA,
            pltpu.SemaphoreType.DMA,
        ],
        compiler_params=pltpu.CompilerParams(use_tc_tiling_on_sc=True),
    )
    def emb_kernel(x_hbm, lut_hbm, out_hbm, idx_v, buf0, buf1, sem0, sem1):
        wid = lax.axis_index("s") * NUM_CORES + lax.axis_index("c")
        row0 = wid * rows_per_w

        bufs = (buf0, buf1)
        sems = (sem0, sem1)

        pltpu.sync_copy(x_hbm.at[pl.ds(row0, rows_per_w)], idx_v)

        def issue(g, buf, sem):
            def fetch16(r, s0, v, lanes):
                for j in lanes:
                    pltpu.async_copy(
                        lut_hbm.at[v[j]],
                        buf.at[r, s0 + j, pl.ds(0, D_MODEL)],
                        sem,
                    )

            for r in range(ROWS_PER_CHUNK):
                xrow = g * ROWS_PER_CHUNK + r

                def body(k, carry, r=r, xrow=xrow):
                    v = idx_v[xrow, pl.ds(k * LANES, LANES)]
                    fetch16(r, k * LANES, v, range(LANES))
                    return carry

                lax.fori_loop(0, seq // LANES, body, 0)
                tail = seq % LANES
                if tail:
                    v = idx_v[xrow, pl.ds(seq - LANES, LANES)]
                    fetch16(r, seq - LANES, v, range(LANES - tail, LANES))

        def drain(buf, sem):
            dummy = pltpu.make_async_copy(
                lut_hbm.at[0], buf.at[0, 0, pl.ds(0, D_MODEL)], sem
            )

            def body(i, carry):
                dummy.wait()
                return carry

            lax.fori_loop(0, ROWS_PER_CHUNK * seq, body, 0)

        def scale(buf):
            def body(s, carry):
                for r in range(ROWS_PER_CHUNK):
                    for c in range(D_MODEL // LANES):
                        sl = pl.ds(c * LANES, LANES)
                        buf[r, s, sl] = buf[r, s, sl] * SCALE
                return carry

            lax.fori_loop(0, seq, body, 0)

        issue(0, bufs[0], sems[0])
        for g in range(num_chunks):
            if g + 1 < num_chunks:
                issue(g + 1, bufs[(g + 1) % 2], sems[(g + 1) % 2])
            drain(bufs[g % 2], sems[g % 2])
            scale(bufs[g % 2])
            pltpu.sync_copy(
                bufs[g % 2],
                out_hbm.at[pl.ds(row0 + g * ROWS_PER_CHUNK, ROWS_PER_CHUNK)],
            )

    return emb_kernel


def kernel(x, lut):
    batch, seq = x.shape
    return _build(batch, seq)(x.astype(jnp.int32), lut)
